# baseline (speedup 1.0000x reference)
import functools

import jax
import jax.numpy as jnp
from jax import lax
from jax.experimental import pallas as pl
from jax.experimental.pallas import tpu as pltpu
from jax.experimental.pallas import tpu_sc as plsc

N = 10000
E = 320000
NC = 2
NS = 16
NW = NC * NS
CC = 128
WCH = 80
EPW = WCH * CC
EP = NW * EPW
NBUF = 5
NPAD = 10240
RPT = NPAD // NS
ZROWS = 128

_sc_mesh = plsc.VectorSubcoreMesh(core_axis_name="c", subcore_axis_name="s")


def _fill_rows(ref, nrows, width, value):
    v = jnp.full((16,), value, jnp.float32)

    def row(r, carry):
        for j in range(width // 16):
            ref[r, pl.ds(j * 16, 16)] = v
        return carry

    lax.fori_loop(0, nrows, row, 0)


def _fill_flat(ref, n, value):
    v = jnp.full((16,), value, jnp.float32)

    def step(k, carry):
        ref[pl.ds(k * 16, 16)] = v
        return carry

    lax.fori_loop(0, n // 16, step, 0)


@functools.partial(
    pl.kernel,
    mesh=_sc_mesh,
    out_type=jax.ShapeDtypeStruct((NC, NPAD, 128), jnp.float32),
    scratch_types=[
        pltpu.VMEM((2, 2, CC), jnp.int32),
        pltpu.VMEM((2, CC, 128), jnp.float32),
        pltpu.VMEM_SHARED((NPAD, 128), jnp.float32),
        pltpu.SemaphoreType.DMA,
        pltpu.SemaphoreType.DMA,
        pltpu.SemaphoreType.DMA,
        pltpu.SemaphoreType.DMA,
    ],
)
def _agg(ed_hbm, hs_hbm, out_hbm, idx, rows, acc, si0, si1, sg0, sg1):
    si = (si0, si1)
    sg = (sg0, sg1)
    cid = lax.axis_index("c")
    sid = lax.axis_index("s")
    wid = sid * NC + cid
    base = wid * WCH

    _fill_rows(rows.at[0], CC, 128, 0.0)
    for k in range(RPT // ZROWS):
        pltpu.sync_copy(rows.at[0], acc.at[pl.ds(sid * RPT + k * ZROWS, ZROWS)])
    plsc.subcore_barrier()

    pltpu.sync_copy(ed_hbm.at[base], idx.at[0])
    pltpu.async_copy(ed_hbm.at[base + 1], idx.at[1], si[1])
    pltpu.async_copy(hs_hbm.at[idx.at[0, 0]], rows.at[0], sg[0])

    def body(j, carry):
        for b in range(2):
            i = 2 * j + b
            nb = b ^ 1
            pltpu.make_async_copy(hs_hbm.at[idx.at[b, 0]], rows.at[b],
                                  sg[b]).wait()
            pltpu.make_async_copy(ed_hbm.at[base], idx.at[nb], si[nb]).wait()
            pltpu.async_copy(hs_hbm.at[idx.at[nb, 0]], rows.at[nb], sg[nb])
            pltpu.sync_copy(rows.at[b], acc.at[idx.at[b, 1]], add=True)
            pltpu.async_copy(ed_hbm.at[base + i + 2], idx.at[b], si[b])
        return carry

    lax.fori_loop(0, (WCH - 2) // 2, body, 0)

    pltpu.make_async_copy(hs_hbm.at[idx.at[0, 0]], rows.at[0], sg[0]).wait()
    pltpu.make_async_copy(ed_hbm.at[base], idx.at[1], si[1]).wait()
    pltpu.async_copy(hs_hbm.at[idx.at[1, 0]], rows.at[1], sg[1])
    pltpu.sync_copy(rows.at[0], acc.at[idx.at[0, 1]], add=True)
    pltpu.make_async_copy(hs_hbm.at[idx.at[1, 0]], rows.at[1], sg[1]).wait()
    pltpu.sync_copy(rows.at[1], acc.at[idx.at[1, 1]], add=True)

    plsc.subcore_barrier()
    pltpu.sync_copy(acc.at[pl.ds(sid * RPT, RPT)],
                    out_hbm.at[cid, pl.ds(sid * RPT, RPT)])


@functools.partial(
    pl.kernel,
    mesh=_sc_mesh,
    out_type=jax.ShapeDtypeStruct((2 * NC * NPAD,), jnp.float32),
    scratch_types=[
        pltpu.VMEM((WCH, 2, CC), jnp.int32),
        pltpu.VMEM((CC,), jnp.float32),
        pltpu.VMEM((RPT,), jnp.float32),
        pltpu.VMEM_SHARED((NPAD,), jnp.float32),
        pltpu.VMEM_SHARED((NPAD,), jnp.float32),
        pltpu.SemaphoreType.DMA,
    ],
)
def _deg(ed_hbm, out_hbm, edv, ones_v, zflat, acc_o, acc_i, sem):
    cid = lax.axis_index("c")
    sid = lax.axis_index("s")
    wid = sid * NC + cid

    pltpu.sync_copy(ed_hbm.at[pl.ds(wid * WCH, WCH)], edv)
    _fill_flat(ones_v, CC, 1.0)
    _fill_flat(zflat, RPT, 0.0)
    pltpu.sync_copy(zflat, acc_o.at[pl.ds(sid * RPT, RPT)])
    pltpu.sync_copy(zflat, acc_i.at[pl.ds(sid * RPT, RPT)])
    plsc.subcore_barrier()

    def body(i, carry):
        pltpu.async_copy(ones_v, acc_o.at[edv.at[i, 0]], sem, add=True)
        pltpu.async_copy(ones_v, acc_i.at[edv.at[i, 1]], sem, add=True)
        return carry

    lax.fori_loop(0, WCH, body, 0)

    def drain(i, carry):
        pltpu.make_async_copy(ones_v, acc_o.at[edv.at[0, 0]], sem).wait()
        pltpu.make_async_copy(ones_v, acc_i.at[edv.at[0, 1]], sem).wait()
        return carry

    lax.fori_loop(0, WCH, drain, 0)
    plsc.subcore_barrier()
    pltpu.sync_copy(acc_o.at[pl.ds(sid * RPT, RPT)],
                    out_hbm.at[pl.ds(cid * NPAD + sid * RPT, RPT)])
    pltpu.sync_copy(acc_i.at[pl.ds(sid * RPT, RPT)],
                    out_hbm.at[pl.ds((2 + cid) * NPAD + sid * RPT, RPT)])


def _norms(degp):
    def body(d_ref, ns_ref, nd_ref):
        for j, o_ref in ((0, ns_ref), (2, nd_ref)):
            d = d_ref[j] + d_ref[j + 1]
            o_ref[...] = jnp.where(d > 0.0, lax.rsqrt(jnp.maximum(d, 1.0)), 0.0)

    return pl.pallas_call(
        body,
        out_shape=(
            jax.ShapeDtypeStruct((NPAD // 128, 128), jnp.float32),
            jax.ShapeDtypeStruct((NPAD // 128, 128), jnp.float32),
        ),
    )(degp)


def _mm_scale(h, W, ns):
    def body(h_ref, w_ref, s_ref, o_ref):
        o_ref[...] = jnp.dot(h_ref[...], w_ref[...],
                             preferred_element_type=jnp.float32) * s_ref[...]

    return pl.pallas_call(
        body,
        out_shape=jax.ShapeDtypeStruct((NPAD, W.shape[1]), jnp.float32),
    )(h, W, ns)


def _mid(p, nd, b, Wn, ns):
    def body(p_ref, nd_ref, b_ref, w_ref, ns_ref, o_ref):
        agg = p_ref[0] + p_ref[1]
        h = agg * nd_ref[...] + b_ref[...]
        h = jnp.maximum(h, 0.0)
        o_ref[...] = jnp.dot(h, w_ref[...],
                             preferred_element_type=jnp.float32) * ns_ref[...]

    return pl.pallas_call(
        body,
        out_shape=jax.ShapeDtypeStruct((NPAD, Wn.shape[1]), jnp.float32),
    )(p, nd, b, Wn, ns)


def _final(p, nd, b, dout):
    def body(p_ref, nd_ref, b_ref, o_ref):
        agg = (p_ref[0] + p_ref[1])[:N, :dout]
        o_ref[...] = agg * nd_ref[...][:N] + b_ref[...]

    return pl.pallas_call(
        body,
        out_shape=jax.ShapeDtypeStruct((N, dout), jnp.float32),
    )(p, nd, b)


def kernel(x, edge_index, W1, b1, W2, b2, W3, b3):
    src = edge_index[0].astype(jnp.int32)
    dst = edge_index[1].astype(jnp.int32)
    pad = EP - E
    src2 = jnp.concatenate([src, jnp.full((pad,), NPAD - 1, jnp.int32)]).reshape(NW * WCH, 1, CC)
    dst2 = jnp.concatenate([dst, jnp.full((pad,), NPAD - 1, jnp.int32)]).reshape(NW * WCH, 1, CC)
    ed = jnp.concatenate([src2, dst2], axis=1)

    dout = W3.shape[1]
    b1 = b1.reshape(1, -1)
    b2 = b2.reshape(1, -1)
    b3 = b3.reshape(1, -1)
    W3p = jnp.pad(W3, ((0, 0), (0, 128 - dout)))

    degp = _deg(ed).reshape(2 * NC, NPAD // 128, 128)
    ns, nd = _norms(degp)
    ns = ns.reshape(NPAD, 1)
    nd = nd.reshape(NPAD, 1)

    xp = jnp.pad(x, ((0, NPAD - N), (0, 0)))
    hs1 = _mm_scale(xp, W1, ns)
    p1 = _agg(ed, hs1)
    hs2 = _mid(p1, nd, b1, W2, ns)
    p2 = _agg(ed, hs2)
    hs3 = _mid(p2, nd, b2, W3p, ns)
    p3 = _agg(ed, hs3)
    return _final(p3, nd, b3, dout)


# speedup vs baseline: 2.7375x; 2.7375x over previous
import functools

import jax
import jax.numpy as jnp
from jax import lax
from jax.experimental import pallas as pl
from jax.experimental.pallas import tpu as pltpu
from jax.experimental.pallas import tpu_sc as plsc

N = 10000
E = 320000
NC = 2
NS = 16
NW = NC * NS
CC = 128
WCH = 80
EPW = WCH * CC
EP = NW * EPW
NBUF = 5
NPAD = 10240
RPT = NPAD // NS
ZROWS = 128

_sc_mesh = plsc.VectorSubcoreMesh(core_axis_name="c", subcore_axis_name="s")


def _fill_rows(ref, nrows, width, value):
    v = jnp.full((16,), value, jnp.float32)

    def row(r, carry):
        for j in range(width // 16):
            ref[r, pl.ds(j * 16, 16)] = v
        return carry

    lax.fori_loop(0, nrows, row, 0)


def _fill_flat(ref, n, value):
    v = jnp.full((16,), value, jnp.float32)

    def step(k, carry):
        ref[pl.ds(k * 16, 16)] = v
        return carry

    lax.fori_loop(0, n // 16, step, 0)


@functools.partial(
    pl.kernel,
    mesh=_sc_mesh,
    out_type=jax.ShapeDtypeStruct((NC, NPAD, 128), jnp.float32),
    scratch_types=[
        pltpu.VMEM((2, 2, CC), jnp.int32),
        pltpu.VMEM((2, CC, 128), jnp.float32),
        pltpu.VMEM_SHARED((NPAD, 128), jnp.float32),
        pltpu.SemaphoreType.DMA,
        pltpu.SemaphoreType.DMA,
        pltpu.SemaphoreType.DMA,
        pltpu.SemaphoreType.DMA,
    ],
)
def _agg(ed_hbm, hs_hbm, out_hbm, idx, rows, acc, si0, si1, sg0, sg1):
    si = (si0, si1)
    sg = (sg0, sg1)
    cid = lax.axis_index("c")
    sid = lax.axis_index("s")
    wid = sid * NC + cid
    base = wid * WCH

    _fill_rows(rows.at[0], CC, 128, 0.0)
    for k in range(RPT // ZROWS):
        pltpu.sync_copy(rows.at[0], acc.at[pl.ds(sid * RPT + k * ZROWS, ZROWS)])
    plsc.subcore_barrier()

    pltpu.sync_copy(ed_hbm.at[base], idx.at[0])
    pltpu.async_copy(ed_hbm.at[base + 1], idx.at[1], si[1])
    pltpu.async_copy(hs_hbm.at[idx.at[0, 0]], rows.at[0], sg[0])

    def body(j, carry):
        for b in range(2):
            i = 2 * j + b
            nb = b ^ 1
            pltpu.make_async_copy(hs_hbm.at[idx.at[b, 0]], rows.at[b],
                                  sg[b]).wait()
            pltpu.make_async_copy(ed_hbm.at[base], idx.at[nb], si[nb]).wait()
            pltpu.async_copy(hs_hbm.at[idx.at[nb, 0]], rows.at[nb], sg[nb])
            pltpu.sync_copy(rows.at[b], acc.at[idx.at[b, 1]], add=True)
            pltpu.async_copy(ed_hbm.at[base + i + 2], idx.at[b], si[b])
        return carry

    lax.fori_loop(0, (WCH - 2) // 2, body, 0)

    pltpu.make_async_copy(hs_hbm.at[idx.at[0, 0]], rows.at[0], sg[0]).wait()
    pltpu.make_async_copy(ed_hbm.at[base], idx.at[1], si[1]).wait()
    pltpu.async_copy(hs_hbm.at[idx.at[1, 0]], rows.at[1], sg[1])
    pltpu.sync_copy(rows.at[0], acc.at[idx.at[0, 1]], add=True)
    pltpu.make_async_copy(hs_hbm.at[idx.at[1, 0]], rows.at[1], sg[1]).wait()
    pltpu.sync_copy(rows.at[1], acc.at[idx.at[1, 1]], add=True)

    plsc.subcore_barrier()
    pltpu.sync_copy(acc.at[pl.ds(sid * RPT, RPT)],
                    out_hbm.at[cid, pl.ds(sid * RPT, RPT)])


@functools.partial(
    pl.kernel,
    mesh=_sc_mesh,
    out_type=jax.ShapeDtypeStruct((2 * NC * NPAD,), jnp.float32),
    scratch_types=[
        pltpu.VMEM((WCH, 2, CC), jnp.int32),
        pltpu.VMEM((CC,), jnp.float32),
        pltpu.VMEM((RPT,), jnp.float32),
        pltpu.VMEM_SHARED((NPAD,), jnp.float32),
        pltpu.VMEM_SHARED((NPAD,), jnp.float32),
        pltpu.SemaphoreType.DMA,
    ],
)
def _deg(ed_hbm, out_hbm, edv, ones_v, zflat, acc_o, acc_i, sem):
    cid = lax.axis_index("c")
    sid = lax.axis_index("s")
    wid = sid * NC + cid

    pltpu.sync_copy(ed_hbm.at[pl.ds(wid * WCH, WCH)], edv)
    _fill_flat(ones_v, CC, 1.0)
    _fill_flat(zflat, RPT, 0.0)
    pltpu.sync_copy(zflat, acc_o.at[pl.ds(sid * RPT, RPT)])
    pltpu.sync_copy(zflat, acc_i.at[pl.ds(sid * RPT, RPT)])
    plsc.subcore_barrier()

    def body(i, carry):
        pltpu.async_copy(ones_v, acc_o.at[edv.at[i, 0]], sem, add=True)
        pltpu.async_copy(ones_v, acc_i.at[edv.at[i, 1]], sem, add=True)
        return carry

    lax.fori_loop(0, WCH, body, 0)

    def drain(i, carry):
        pltpu.make_async_copy(ones_v, acc_o.at[edv.at[0, 0]], sem).wait()
        pltpu.make_async_copy(ones_v, acc_i.at[edv.at[0, 1]], sem).wait()
        return carry

    lax.fori_loop(0, WCH, drain, 0)
    plsc.subcore_barrier()
    pltpu.sync_copy(acc_o.at[pl.ds(sid * RPT, RPT)],
                    out_hbm.at[pl.ds(cid * NPAD + sid * RPT, RPT)])
    pltpu.sync_copy(acc_i.at[pl.ds(sid * RPT, RPT)],
                    out_hbm.at[pl.ds((2 + cid) * NPAD + sid * RPT, RPT)])


def _norms(degp):
    def body(d_ref, ns_ref, nd_ref):
        for j, o_ref in ((0, ns_ref), (2, nd_ref)):
            d = d_ref[j] + d_ref[j + 1]
            o_ref[...] = jnp.where(d > 0.0, lax.rsqrt(jnp.maximum(d, 1.0)), 0.0)

    return pl.pallas_call(
        body,
        out_shape=(
            jax.ShapeDtypeStruct((NPAD // 128, 128), jnp.float32),
            jax.ShapeDtypeStruct((NPAD // 128, 128), jnp.float32),
        ),
    )(degp)


def _mm_scale(h, W, ns):
    def body(h_ref, w_ref, s_ref, o_ref):
        o_ref[...] = jnp.dot(h_ref[...], w_ref[...],
                             preferred_element_type=jnp.float32) * s_ref[...]

    return pl.pallas_call(
        body,
        out_shape=jax.ShapeDtypeStruct((NPAD, W.shape[1]), jnp.float32),
    )(h, W, ns)


def _mid(p, nd, b, Wn, ns):
    def body(p_ref, nd_ref, b_ref, w_ref, ns_ref, o_ref):
        agg = p_ref[0] + p_ref[1]
        h = agg * nd_ref[...] + b_ref[...]
        h = jnp.maximum(h, 0.0)
        o_ref[...] = jnp.dot(h, w_ref[...],
                             preferred_element_type=jnp.float32) * ns_ref[...]

    return pl.pallas_call(
        body,
        out_shape=jax.ShapeDtypeStruct((NPAD, Wn.shape[1]), jnp.float32),
    )(p, nd, b, Wn, ns)


def _final(p, nd, b, dout):
    def body(p_ref, nd_ref, b_ref, o_ref):
        agg = (p_ref[0] + p_ref[1])[:N, :dout]
        o_ref[...] = agg * nd_ref[...][:N] + b_ref[...]

    return pl.pallas_call(
        body,
        out_shape=jax.ShapeDtypeStruct((N, dout), jnp.float32),
    )(p, nd, b)


def kernel(x, edge_index, W1, b1, W2, b2, W3, b3):
    src = edge_index[0].astype(jnp.int32)
    dst = edge_index[1].astype(jnp.int32)
    pad = EP - E
    deadrow = N + (jnp.arange(pad, dtype=jnp.int32) % (NPAD - N))
    src2 = jnp.concatenate([src, deadrow]).reshape(NW * WCH, 1, CC)
    dst2 = jnp.concatenate([dst, deadrow]).reshape(NW * WCH, 1, CC)
    ed = jnp.concatenate([src2, dst2], axis=1)

    dout = W3.shape[1]
    b1 = b1.reshape(1, -1)
    b2 = b2.reshape(1, -1)
    b3 = b3.reshape(1, -1)
    W3p = jnp.pad(W3, ((0, 0), (0, 128 - dout)))

    degp = _deg(ed).reshape(2 * NC, NPAD // 128, 128)
    ns, nd = _norms(degp)
    ns = ns.reshape(NPAD, 1)
    nd = nd.reshape(NPAD, 1)

    xp = jnp.pad(x, ((0, NPAD - N), (0, 0)))
    hs1 = _mm_scale(xp, W1, ns)
    p1 = _agg(ed, hs1)
    hs2 = _mid(p1, nd, b1, W2, ns)
    p2 = _agg(ed, hs2)
    hs3 = _mid(p2, nd, b2, W3p, ns)
    p3 = _agg(ed, hs3)
    return _final(p3, nd, b3, dout)


# speedup vs baseline: 2.9418x; 1.0746x over previous
import functools

import jax
import jax.numpy as jnp
from jax import lax
from jax.experimental import pallas as pl
from jax.experimental.pallas import tpu as pltpu
from jax.experimental.pallas import tpu_sc as plsc

N = 10000
E = 320000
NC = 2
NS = 16
NW = NC * NS
CC = 120
WCH = 84
EPW = WCH * CC
EP = NW * EPW
NBUF = 5
NPAD = 10240
RPT = NPAD // NS
ZROWS = 128

_sc_mesh = plsc.VectorSubcoreMesh(core_axis_name="c", subcore_axis_name="s")


def _fill_rows(ref, nrows, width, value):
    v = jnp.full((16,), value, jnp.float32)

    def row(r, carry):
        for j in range(width // 16):
            ref[r, pl.ds(j * 16, 16)] = v
        return carry

    lax.fori_loop(0, nrows, row, 0)


def _fill_flat(ref, n, value):
    v = jnp.full((16,), value, jnp.float32)

    def step(k, carry):
        ref[pl.ds(k * 16, 16)] = v
        return carry

    lax.fori_loop(0, n // 16, step, 0)


@functools.partial(
    pl.kernel,
    mesh=_sc_mesh,
    out_type=jax.ShapeDtypeStruct((NC, NPAD, 128), jnp.float32),
    scratch_types=[
        pltpu.VMEM((3, 2, CC), jnp.int32),
        pltpu.VMEM((3, CC, 128), jnp.float32),
        pltpu.VMEM_SHARED((NPAD, 128), jnp.float32),
        pltpu.SemaphoreType.DMA,
        pltpu.SemaphoreType.DMA,
        pltpu.SemaphoreType.DMA,
        pltpu.SemaphoreType.DMA,
        pltpu.SemaphoreType.DMA,
        pltpu.SemaphoreType.DMA,
    ],
)
def _agg(ed_hbm, hs_hbm, out_hbm, idx, rows, acc, si0, si1, si2, sg0, sg1, sg2):
    si = (si0, si1, si2)
    sg = (sg0, sg1, sg2)
    cid = lax.axis_index("c")
    sid = lax.axis_index("s")
    wid = sid * NC + cid
    base = wid * WCH

    _fill_rows(rows.at[0], 80, 128, 0.0)
    for k in range(RPT // 80):
        pltpu.sync_copy(rows.at[0, pl.ds(0, 80)],
                        acc.at[pl.ds(sid * RPT + k * 80, 80)])
    plsc.subcore_barrier()

    pltpu.sync_copy(ed_hbm.at[base], idx.at[0])
    pltpu.async_copy(ed_hbm.at[base + 1], idx.at[1], si[1])
    pltpu.async_copy(ed_hbm.at[base + 2], idx.at[2], si[2])
    pltpu.async_copy(hs_hbm.at[idx.at[0, 0]], rows.at[0], sg[0])
    pltpu.make_async_copy(ed_hbm.at[base], idx.at[1], si[1]).wait()
    pltpu.async_copy(hs_hbm.at[idx.at[1, 0]], rows.at[1], sg[1])

    def body(j, carry):
        for b in range(3):
            i = 3 * j + b
            b2 = (b + 2) % 3
            pltpu.make_async_copy(hs_hbm.at[idx.at[b, 0]], rows.at[b],
                                  sg[b]).wait()
            pltpu.make_async_copy(ed_hbm.at[base], idx.at[b2], si[b2]).wait()
            pltpu.async_copy(hs_hbm.at[idx.at[b2, 0]], rows.at[b2], sg[b2])
            pltpu.sync_copy(rows.at[b], acc.at[idx.at[b, 1]], add=True)
            pltpu.async_copy(ed_hbm.at[base + i + 3], idx.at[b], si[b])
        return carry

    lax.fori_loop(0, (WCH - 3) // 3, body, 0)

    pltpu.make_async_copy(hs_hbm.at[idx.at[0, 0]], rows.at[0], sg[0]).wait()
    pltpu.make_async_copy(ed_hbm.at[base], idx.at[2], si[2]).wait()
    pltpu.async_copy(hs_hbm.at[idx.at[2, 0]], rows.at[2], sg[2])
    pltpu.sync_copy(rows.at[0], acc.at[idx.at[0, 1]], add=True)
    pltpu.make_async_copy(hs_hbm.at[idx.at[1, 0]], rows.at[1], sg[1]).wait()
    pltpu.sync_copy(rows.at[1], acc.at[idx.at[1, 1]], add=True)
    pltpu.make_async_copy(hs_hbm.at[idx.at[2, 0]], rows.at[2], sg[2]).wait()
    pltpu.sync_copy(rows.at[2], acc.at[idx.at[2, 1]], add=True)

    plsc.subcore_barrier()
    pltpu.sync_copy(acc.at[pl.ds(sid * RPT, RPT)],
                    out_hbm.at[cid, pl.ds(sid * RPT, RPT)])


@functools.partial(
    pl.kernel,
    mesh=_sc_mesh,
    out_type=jax.ShapeDtypeStruct((2 * NC * NPAD,), jnp.float32),
    scratch_types=[
        pltpu.VMEM((WCH, 2, CC), jnp.int32),
        pltpu.VMEM((CC,), jnp.float32),
        pltpu.VMEM((RPT,), jnp.float32),
        pltpu.VMEM_SHARED((NPAD,), jnp.float32),
        pltpu.VMEM_SHARED((NPAD,), jnp.float32),
        pltpu.SemaphoreType.DMA,
    ],
)
def _deg(ed_hbm, out_hbm, edv, ones_v, zflat, acc_o, acc_i, sem):
    cid = lax.axis_index("c")
    sid = lax.axis_index("s")
    wid = sid * NC + cid

    pltpu.sync_copy(ed_hbm.at[pl.ds(wid * WCH, WCH)], edv)
    _fill_flat(ones_v, CC, 1.0)
    _fill_flat(zflat, RPT, 0.0)
    pltpu.sync_copy(zflat, acc_o.at[pl.ds(sid * RPT, RPT)])
    pltpu.sync_copy(zflat, acc_i.at[pl.ds(sid * RPT, RPT)])
    plsc.subcore_barrier()

    def body(i, carry):
        pltpu.async_copy(ones_v, acc_o.at[edv.at[i, 0]], sem, add=True)
        pltpu.async_copy(ones_v, acc_i.at[edv.at[i, 1]], sem, add=True)
        return carry

    lax.fori_loop(0, WCH, body, 0)

    def drain(i, carry):
        pltpu.make_async_copy(ones_v, acc_o.at[edv.at[0, 0]], sem).wait()
        pltpu.make_async_copy(ones_v, acc_i.at[edv.at[0, 1]], sem).wait()
        return carry

    lax.fori_loop(0, WCH, drain, 0)
    plsc.subcore_barrier()
    pltpu.sync_copy(acc_o.at[pl.ds(sid * RPT, RPT)],
                    out_hbm.at[pl.ds(cid * NPAD + sid * RPT, RPT)])
    pltpu.sync_copy(acc_i.at[pl.ds(sid * RPT, RPT)],
                    out_hbm.at[pl.ds((2 + cid) * NPAD + sid * RPT, RPT)])


def _norms(degp):
    def body(d_ref, ns_ref, nd_ref):
        for j, o_ref in ((0, ns_ref), (2, nd_ref)):
            d = d_ref[j] + d_ref[j + 1]
            o_ref[...] = jnp.where(d > 0.0, lax.rsqrt(jnp.maximum(d, 1.0)), 0.0)

    return pl.pallas_call(
        body,
        out_shape=(
            jax.ShapeDtypeStruct((NPAD // 128, 128), jnp.float32),
            jax.ShapeDtypeStruct((NPAD // 128, 128), jnp.float32),
        ),
    )(degp)


def _mm_scale(h, W, ns):
    def body(h_ref, w_ref, s_ref, o_ref):
        o_ref[...] = jnp.dot(h_ref[...], w_ref[...],
                             preferred_element_type=jnp.float32) * s_ref[...]

    return pl.pallas_call(
        body,
        out_shape=jax.ShapeDtypeStruct((NPAD, W.shape[1]), jnp.float32),
    )(h, W, ns)


def _mid(p, nd, b, Wn, ns):
    def body(p_ref, nd_ref, b_ref, w_ref, ns_ref, o_ref):
        agg = p_ref[0] + p_ref[1]
        h = agg * nd_ref[...] + b_ref[...]
        h = jnp.maximum(h, 0.0)
        o_ref[...] = jnp.dot(h, w_ref[...],
                             preferred_element_type=jnp.float32) * ns_ref[...]

    return pl.pallas_call(
        body,
        out_shape=jax.ShapeDtypeStruct((NPAD, Wn.shape[1]), jnp.float32),
    )(p, nd, b, Wn, ns)


def _final(p, nd, b, dout):
    def body(p_ref, nd_ref, b_ref, o_ref):
        agg = (p_ref[0] + p_ref[1])[:N, :dout]
        o_ref[...] = agg * nd_ref[...][:N] + b_ref[...]

    return pl.pallas_call(
        body,
        out_shape=jax.ShapeDtypeStruct((N, dout), jnp.float32),
    )(p, nd, b)


def kernel(x, edge_index, W1, b1, W2, b2, W3, b3):
    src = edge_index[0].astype(jnp.int32)
    dst = edge_index[1].astype(jnp.int32)
    pad = EP - E
    deadrow = N + (jnp.arange(pad, dtype=jnp.int32) % (NPAD - N))
    src2 = jnp.concatenate([src, deadrow]).reshape(NW * WCH, 1, CC)
    dst2 = jnp.concatenate([dst, deadrow]).reshape(NW * WCH, 1, CC)
    ed = jnp.concatenate([src2, dst2], axis=1)

    dout = W3.shape[1]
    b1 = b1.reshape(1, -1)
    b2 = b2.reshape(1, -1)
    b3 = b3.reshape(1, -1)
    W3p = jnp.pad(W3, ((0, 0), (0, 128 - dout)))

    degp = _deg(ed).reshape(2 * NC, NPAD // 128, 128)
    ns, nd = _norms(degp)
    ns = ns.reshape(NPAD, 1)
    nd = nd.reshape(NPAD, 1)

    xp = jnp.pad(x, ((0, NPAD - N), (0, 0)))
    hs1 = _mm_scale(xp, W1, ns)
    p1 = _agg(ed, hs1)
    hs2 = _mid(p1, nd, b1, W2, ns)
    p2 = _agg(ed, hs2)
    hs3 = _mid(p2, nd, b2, W3p, ns)
    p3 = _agg(ed, hs3)
    return _final(p3, nd, b3, dout)


# speedup vs baseline: 2.9458x; 1.0014x over previous
import functools

import jax
import jax.numpy as jnp
from jax import lax
from jax.experimental import pallas as pl
from jax.experimental.pallas import tpu as pltpu
from jax.experimental.pallas import tpu_sc as plsc

N = 10000
E = 320000
NC = 2
NS = 16
NW = NC * NS
CC = 128
WCH = 81
EPW = WCH * CC
EP = NW * EPW
NPAD = 10112
RPT = NPAD // NS
DEGPAD = 10240
RPTD = DEGPAD // NS

_sc_mesh = plsc.VectorSubcoreMesh(core_axis_name="c", subcore_axis_name="s")


def _fill_rows(ref, nrows, width, value):
    v = jnp.full((16,), value, jnp.float32)

    def row(r, carry):
        for j in range(width // 16):
            ref[r, pl.ds(j * 16, 16)] = v
        return carry

    lax.fori_loop(0, nrows, row, 0)


def _fill_flat(ref, n, value):
    v = jnp.full((16,), value, jnp.float32)

    def step(k, carry):
        ref[pl.ds(k * 16, 16)] = v
        return carry

    lax.fori_loop(0, n // 16, step, 0)
    if n % 16:
        ref[pl.ds(n - 16, 16)] = v


@functools.partial(
    pl.kernel,
    mesh=_sc_mesh,
    out_type=jax.ShapeDtypeStruct((NC, NPAD, 128), jnp.float32),
    scratch_types=[
        pltpu.VMEM((3, 2, CC), jnp.int32),
        pltpu.VMEM((3, CC, 128), jnp.float32),
        pltpu.VMEM_SHARED((NPAD, 128), jnp.float32),
        pltpu.SemaphoreType.DMA,
        pltpu.SemaphoreType.DMA,
        pltpu.SemaphoreType.DMA,
        pltpu.SemaphoreType.DMA,
        pltpu.SemaphoreType.DMA,
        pltpu.SemaphoreType.DMA,
    ],
)
def _agg(ed_hbm, hs_hbm, out_hbm, idx, rows, acc, si0, si1, si2, sg0, sg1, sg2):
    si = (si0, si1, si2)
    sg = (sg0, sg1, sg2)
    cid = lax.axis_index("c")
    sid = lax.axis_index("s")
    wid = sid * NC + cid
    base = wid * WCH

    _fill_rows(rows.at[0], CC, 128, 0.0)
    for k in range(4):
        pltpu.sync_copy(rows.at[0], acc.at[pl.ds(sid * RPT + k * 128, 128)])
    pltpu.sync_copy(rows.at[0, pl.ds(0, RPT - 512)],
                    acc.at[pl.ds(sid * RPT + 512, RPT - 512)])
    plsc.subcore_barrier()

    pltpu.sync_copy(ed_hbm.at[base], idx.at[0])
    pltpu.async_copy(ed_hbm.at[base + 1], idx.at[1], si[1])
    pltpu.async_copy(ed_hbm.at[base + 2], idx.at[2], si[2])
    pltpu.async_copy(hs_hbm.at[idx.at[0, 0]], rows.at[0], sg[0])
    pltpu.make_async_copy(ed_hbm.at[base], idx.at[1], si[1]).wait()
    pltpu.async_copy(hs_hbm.at[idx.at[1, 0]], rows.at[1], sg[1])

    def body(j, carry):
        for b in range(3):
            i = 3 * j + b
            b2 = (b + 2) % 3
            pltpu.make_async_copy(hs_hbm.at[idx.at[b, 0]], rows.at[b],
                                  sg[b]).wait()
            pltpu.make_async_copy(ed_hbm.at[base], idx.at[b2], si[b2]).wait()
            pltpu.async_copy(hs_hbm.at[idx.at[b2, 0]], rows.at[b2], sg[b2])
            pltpu.sync_copy(rows.at[b], acc.at[idx.at[b, 1]], add=True)
            pltpu.async_copy(ed_hbm.at[base + i + 3], idx.at[b], si[b])
        return carry

    lax.fori_loop(0, (WCH - 3) // 3, body, 0)

    pltpu.make_async_copy(hs_hbm.at[idx.at[0, 0]], rows.at[0], sg[0]).wait()
    pltpu.make_async_copy(ed_hbm.at[base], idx.at[2], si[2]).wait()
    pltpu.async_copy(hs_hbm.at[idx.at[2, 0]], rows.at[2], sg[2])
    pltpu.sync_copy(rows.at[0], acc.at[idx.at[0, 1]], add=True)
    pltpu.make_async_copy(hs_hbm.at[idx.at[1, 0]], rows.at[1], sg[1]).wait()
    pltpu.sync_copy(rows.at[1], acc.at[idx.at[1, 1]], add=True)
    pltpu.make_async_copy(hs_hbm.at[idx.at[2, 0]], rows.at[2], sg[2]).wait()
    pltpu.sync_copy(rows.at[2], acc.at[idx.at[2, 1]], add=True)

    plsc.subcore_barrier()
    pltpu.sync_copy(acc.at[pl.ds(sid * RPT, RPT)],
                    out_hbm.at[cid, pl.ds(sid * RPT, RPT)])


@functools.partial(
    pl.kernel,
    mesh=_sc_mesh,
    out_type=jax.ShapeDtypeStruct((2 * NC * DEGPAD,), jnp.float32),
    scratch_types=[
        pltpu.VMEM((WCH, 2, CC), jnp.int32),
        pltpu.VMEM((CC,), jnp.float32),
        pltpu.VMEM((RPTD,), jnp.float32),
        pltpu.VMEM_SHARED((DEGPAD,), jnp.float32),
        pltpu.VMEM_SHARED((DEGPAD,), jnp.float32),
        pltpu.SemaphoreType.DMA,
    ],
)
def _deg(ed_hbm, out_hbm, edv, ones_v, zflat, acc_o, acc_i, sem):
    cid = lax.axis_index("c")
    sid = lax.axis_index("s")
    wid = sid * NC + cid

    pltpu.sync_copy(ed_hbm.at[pl.ds(wid * WCH, WCH)], edv)
    _fill_flat(ones_v, CC, 1.0)
    _fill_flat(zflat, RPTD, 0.0)
    pltpu.sync_copy(zflat, acc_o.at[pl.ds(sid * RPTD, RPTD)])
    pltpu.sync_copy(zflat, acc_i.at[pl.ds(sid * RPTD, RPTD)])
    plsc.subcore_barrier()

    def body(i, carry):
        pltpu.async_copy(ones_v, acc_o.at[edv.at[i, 0]], sem, add=True)
        pltpu.async_copy(ones_v, acc_i.at[edv.at[i, 1]], sem, add=True)
        return carry

    lax.fori_loop(0, WCH, body, 0)

    def drain(i, carry):
        pltpu.make_async_copy(ones_v, acc_o.at[edv.at[0, 0]], sem).wait()
        pltpu.make_async_copy(ones_v, acc_i.at[edv.at[0, 1]], sem).wait()
        return carry

    lax.fori_loop(0, WCH, drain, 0)
    plsc.subcore_barrier()
    pltpu.sync_copy(acc_o.at[pl.ds(sid * RPTD, RPTD)],
                    out_hbm.at[pl.ds(cid * DEGPAD + sid * RPTD, RPTD)])
    pltpu.sync_copy(acc_i.at[pl.ds(sid * RPTD, RPTD)],
                    out_hbm.at[pl.ds((2 + cid) * DEGPAD + sid * RPTD, RPTD)])


def _norms(degp):
    def body(d_ref, ns_ref, nd_ref):
        for j, o_ref in ((0, ns_ref), (2, nd_ref)):
            d = d_ref[j] + d_ref[j + 1]
            o_ref[...] = jnp.where(d > 0.0, lax.rsqrt(jnp.maximum(d, 1.0)), 0.0)

    return pl.pallas_call(
        body,
        out_shape=(
            jax.ShapeDtypeStruct((DEGPAD // 128, 128), jnp.float32),
            jax.ShapeDtypeStruct((DEGPAD // 128, 128), jnp.float32),
        ),
    )(degp)


def _mm_scale(h, W, ns):
    def body(h_ref, w_ref, s_ref, o_ref):
        o_ref[...] = jnp.dot(h_ref[...], w_ref[...],
                             preferred_element_type=jnp.float32) * s_ref[...]

    return pl.pallas_call(
        body,
        out_shape=jax.ShapeDtypeStruct((NPAD, W.shape[1]), jnp.float32),
    )(h, W, ns)


def _mid(p, nd, b, Wn, ns):
    def body(p_ref, nd_ref, b_ref, w_ref, ns_ref, o_ref):
        agg = p_ref[0] + p_ref[1]
        h = agg * nd_ref[...] + b_ref[...]
        h = jnp.maximum(h, 0.0)
        o_ref[...] = jnp.dot(h, w_ref[...],
                             preferred_element_type=jnp.float32) * ns_ref[...]

    return pl.pallas_call(
        body,
        out_shape=jax.ShapeDtypeStruct((NPAD, Wn.shape[1]), jnp.float32),
    )(p, nd, b, Wn, ns)


def _final(p, nd, b, dout):
    def body(p_ref, nd_ref, b_ref, o_ref):
        agg = (p_ref[0] + p_ref[1])[:N, :dout]
        o_ref[...] = agg * nd_ref[...][:N] + b_ref[...]

    return pl.pallas_call(
        body,
        out_shape=jax.ShapeDtypeStruct((N, dout), jnp.float32),
    )(p, nd, b)


def kernel(x, edge_index, W1, b1, W2, b2, W3, b3):
    src = edge_index[0].astype(jnp.int32)
    dst = edge_index[1].astype(jnp.int32)
    pad = EP - E
    deadrow = N + (jnp.arange(pad, dtype=jnp.int32) % (NPAD - N))
    src2 = jnp.concatenate([src, deadrow]).reshape(NW * WCH, 1, CC)
    dst2 = jnp.concatenate([dst, deadrow]).reshape(NW * WCH, 1, CC)
    ed = jnp.concatenate([src2, dst2], axis=1)

    dout = W3.shape[1]
    b1 = b1.reshape(1, -1)
    b2 = b2.reshape(1, -1)
    b3 = b3.reshape(1, -1)
    W3p = jnp.pad(W3, ((0, 0), (0, 128 - dout)))

    degp = _deg(ed).reshape(2 * NC, DEGPAD // 128, 128)
    ns, nd = _norms(degp)
    ns = ns.reshape(DEGPAD, 1)[:NPAD]
    nd = nd.reshape(DEGPAD, 1)[:NPAD]

    xp = jnp.pad(x, ((0, NPAD - N), (0, 0)))
    hs1 = _mm_scale(xp, W1, ns)
    p1 = _agg(ed, hs1)
    hs2 = _mid(p1, nd, b1, W2, ns)
    p2 = _agg(ed, hs2)
    hs3 = _mid(p2, nd, b2, W3p, ns)
    p3 = _agg(ed, hs3)
    return _final(p3, nd, b3, dout)
